# baseline (speedup 1.0000x reference)
import functools

import jax
import jax.numpy as jnp
from jax import lax
from jax.experimental import pallas as pl
from jax.experimental.pallas import tpu as pltpu
from jax.experimental.pallas import tpu_sc as plsc

N = 8192
K = 20
R = 256
C = 512
NCT = N // C
PB_E = 256
PB_H = 512
INT_MAX = 2**31 - 1


def _knn_body(xr_ref, xc_ref, br_ref, bc_ref, wdt_ref, wvt_ref, bdv_ref,
              idx_ref, u_ref, v_ref, bd_scr, bi_scr):
    j = pl.program_id(1)
    xr = xr_ref[...]

    @pl.when(j == 0)
    def _init():
        bd_scr[...] = jnp.full((R, 32), jnp.inf, jnp.float32)
        bi_scr[...] = jnp.zeros((R, 32), jnp.int32)
        u_ref[...] = (jnp.dot(xr, wdt_ref[...],
                              preferred_element_type=jnp.float32)
                      + bdv_ref[...])
        v_ref[...] = jnp.dot(xr, wvt_ref[...],
                             preferred_element_type=jnp.float32)

    br = br_ref[...]
    bc = bc_ref[...]
    valid = (jnp.min(bc) <= jnp.max(br)) & (jnp.max(bc) >= jnp.min(br))

    @pl.when(valid)
    def _tile():
        xc = xc_ref[...]
        sqc = jnp.sum(xc * xc, axis=1, keepdims=True)
        a = jnp.concatenate([xr, jnp.ones((R, 1), jnp.float32)], axis=1)
        bmat = jnp.concatenate([-2.0 * xc, sqc], axis=1)
        rank = lax.dot_general(a, bmat, (((1,), (1,)), ((), ())),
                               preferred_element_type=jnp.float32)
        mask = br != bc
        rank = jnp.where(mask, jnp.inf, rank)
        ids = lax.broadcasted_iota(jnp.int32, (R, C), 1) + j * C
        cur = jnp.concatenate([rank, bd_scr[...]], axis=1)
        idc = jnp.concatenate([ids, bi_scr[...]], axis=1)
        ms, mis = [], []
        for _ in range(K):
            m = jnp.min(cur, axis=1, keepdims=True)
            sel = jnp.where(cur == m, idc, INT_MAX)
            mi = jnp.min(sel, axis=1, keepdims=True)
            ms.append(m)
            mis.append(mi)
            cur = jnp.where(sel == mi, jnp.inf, cur)
        bd_scr[:, 0:K] = jnp.concatenate(ms, axis=1)
        bi_scr[:, 0:K] = jnp.concatenate(mis, axis=1)

    @pl.when(j == NCT - 1)
    def _fin():
        idx_ref[...] = bi_scr[:, 0:K]


def _knn(x, batch2d_r, batch2d_c, wdt, wvt, bdv):
    f = x.shape[1]
    fu = wdt.shape[1]
    return pl.pallas_call(
        _knn_body,
        grid=(N // R, NCT),
        in_specs=[
            pl.BlockSpec((R, f), lambda i, j: (i, 0)),
            pl.BlockSpec((C, f), lambda i, j: (j, 0)),
            pl.BlockSpec((R, 1), lambda i, j: (i, 0)),
            pl.BlockSpec((1, C), lambda i, j: (0, j)),
            pl.BlockSpec((f, fu), lambda i, j: (0, 0)),
            pl.BlockSpec((f, fu), lambda i, j: (0, 0)),
            pl.BlockSpec((1, fu), lambda i, j: (0, 0)),
        ],
        out_specs=[
            pl.BlockSpec((R, K), lambda i, j: (i, 0)),
            pl.BlockSpec((R, fu), lambda i, j: (i, 0)),
            pl.BlockSpec((R, fu), lambda i, j: (i, 0)),
        ],
        out_shape=[
            jax.ShapeDtypeStruct((N, K), jnp.int32),
            jax.ShapeDtypeStruct((N, fu), jnp.float32),
            jax.ShapeDtypeStruct((N, fu), jnp.float32),
        ],
        scratch_shapes=[
            pltpu.VMEM((R, 32), jnp.float32),
            pltpu.VMEM((R, 32), jnp.int32),
        ],
    )(x, x, batch2d_r, batch2d_c, wdt, wvt, bdv)


def _sc_gather(table, idx_flat):
    f = table.shape[1]
    e = idx_flat.shape[0]
    nw = 32
    e_per_w = e // nw
    ch = 128
    n_ch = e_per_w // ch
    mesh = plsc.VectorSubcoreMesh(core_axis_name="c", subcore_axis_name="s")

    @functools.partial(
        pl.kernel, mesh=mesh,
        compiler_params=pltpu.CompilerParams(use_tc_tiling_on_sc=False),
        out_type=jax.ShapeDtypeStruct((e, f), jnp.float32),
        scratch_types=[
            pltpu.VMEM((ch,), jnp.int32),
            pltpu.VMEM((ch, f), jnp.float32),
            pltpu.SemaphoreType.DMA,
        ],
    )
    def k(table_hbm, idx_hbm, out_hbm, idx_v, rows_v, sem):
        wid = lax.axis_index("s") * 2 + lax.axis_index("c")
        base = wid * e_per_w

        def body(ci, carry):
            off = base + ci * ch
            pltpu.sync_copy(idx_hbm.at[pl.ds(off, ch)], idx_v)
            pltpu.async_copy(table_hbm.at[idx_v], rows_v, sem).wait()
            pltpu.sync_copy(rows_v, out_hbm.at[pl.ds(off, ch)])
            return carry

        lax.fori_loop(0, n_ch, body, 0)

    return k(table, idx_flat)


def _edge_body(vg_ref, u_ref, w2_ref, b2_ref, out_ref):
    u = u_ref[...]
    w2 = w2_ref[...]
    b2 = b2_ref[...]
    f = u.shape[1]
    acc = jnp.full((PB_E, f), -jnp.inf, jnp.float32)
    for kk in range(K):
        vk = vg_ref[:, kk, :]
        h1 = jnp.maximum(u + vk, 0.0)
        h2 = jnp.dot(h1, w2, preferred_element_type=jnp.float32) + b2
        acc = jnp.maximum(acc, h2)
    out_ref[...] = jnp.maximum(acc, 0.0)


def _edge_mlp(vg3, u, w2t, b2v):
    f = u.shape[1]
    return pl.pallas_call(
        _edge_body,
        grid=(N // PB_E,),
        in_specs=[
            pl.BlockSpec((PB_E, K, f), lambda i: (i, 0, 0)),
            pl.BlockSpec((PB_E, f), lambda i: (i, 0)),
            pl.BlockSpec((f, f), lambda i: (0, 0)),
            pl.BlockSpec((1, f), lambda i: (0, 0)),
        ],
        out_specs=pl.BlockSpec((PB_E, f), lambda i: (i, 0)),
        out_shape=jax.ShapeDtypeStruct((N, f), jnp.float32),
    )(vg3, u, w2t, b2v)


def _head_body(x1_ref, x2_ref, x3_ref, wg1_ref, bg1_ref, wg2_ref, bg2_ref,
               we1a_ref, we1b_ref, be1_ref, we2_ref, be2_ref, we3_ref,
               be3_ref, wn_ref, out_ref):
    c = jnp.concatenate([x1_ref[...], x2_ref[...], x3_ref[...]], axis=1)
    g1 = jnp.maximum(jnp.dot(c, wg1_ref[...],
                             preferred_element_type=jnp.float32)
                     + bg1_ref[...], 0.0)
    g2 = jnp.maximum(jnp.dot(g1, wg2_ref[...],
                             preferred_element_type=jnp.float32)
                     + bg2_ref[...], 0.0)
    e1 = jnp.maximum(jnp.dot(c, we1a_ref[...],
                             preferred_element_type=jnp.float32)
                     + jnp.dot(g2, we1b_ref[...],
                               preferred_element_type=jnp.float32)
                     + be1_ref[...], 0.0)
    e2 = jnp.maximum(jnp.dot(e1, we2_ref[...],
                             preferred_element_type=jnp.float32)
                     + be2_ref[...], 0.0)
    h = jnp.dot(e2, we3_ref[...], preferred_element_type=jnp.float32) \
        + be3_ref[...]
    nrm = jnp.sqrt(jnp.sum(h * h, axis=1, keepdims=True))
    xn = h / jnp.maximum(nrm, 1e-12)
    out_ref[...] = jnp.dot(xn, wn_ref[...],
                           preferred_element_type=jnp.float32) * 30.0


def _head(x1, x2, x3, ws):
    full = lambda a: pl.BlockSpec(a.shape, lambda i: tuple(0 for _ in a.shape))
    return pl.pallas_call(
        _head_body,
        grid=(N // PB_H,),
        in_specs=[
            pl.BlockSpec((PB_H, 64), lambda i: (i, 0)),
            pl.BlockSpec((PB_H, 64), lambda i: (i, 0)),
            pl.BlockSpec((PB_H, 128), lambda i: (i, 0)),
        ] + [full(w) for w in ws],
        out_specs=pl.BlockSpec((PB_H, 3), lambda i: (i, 0)),
        out_shape=jax.ShapeDtypeStruct((N, 3), jnp.float32),
    )(x1, x2, x3, *ws)


_BN_C = 1.0 / float(1.0 + 1e-5) ** 0.5


def _fold(lin, bn):
    s = bn["g"] * _BN_C
    return lin["W"] * s[:, None], lin["b"] * s + bn["bt"]


def _edge_params(l1, bn1, l2, bn2, fin):
    w1, b1 = _fold(l1, bn1)
    wa, wb = w1[:, :fin], w1[:, fin:]
    wdt = (wa - wb).T
    wvt = wb.T
    w2, b2 = _fold(l2, bn2)
    return wdt, wvt, b1[None, :], w2.T, b2[None, :]


def kernel(pos, batch, label, params):
    p = params
    br = batch.reshape(N, 1)
    bc = batch.reshape(1, N)

    wdt1, wvt1, bd1, w2t1, b2v1 = _edge_params(
        p["c1l1"], p["c1bn1"], p["c1l2"], p["c1bn2"], 3)
    wdt2, wvt2, bd2, w2t2, b2v2 = _edge_params(
        p["c2l1"], p["c2bn1"], p["c2l2"], p["c2bn2"], 64)
    wdt3, wvt3, bd3, w2t3, b2v3 = _edge_params(
        p["c3l1"], p["c3bn1"], p["c3l2"], p["c3bn2"], 64)

    idx1, u1, v1 = _knn(pos, br, bc, wdt1, wvt1, bd1)
    vg1 = _sc_gather(v1, idx1.reshape(-1)).reshape(N, K, 64)
    x1 = _edge_mlp(vg1, u1, w2t1, b2v1)

    idx2, u2, v2 = _knn(x1, br, bc, wdt2, wvt2, bd2)
    vg2 = _sc_gather(v2, idx2.reshape(-1)).reshape(N, K, 64)
    x2 = _edge_mlp(vg2, u2, w2t2, b2v2)

    idx3, u3, v3 = _knn(x2, br, bc, wdt3, wvt3, bd3)
    vg3 = _sc_gather(v3, idx3.reshape(-1)).reshape(N, K, 128)
    x3 = _edge_mlp(vg3, u3, w2t3, b2v3)

    wg1, bg1 = _fold(p["g1"], p["gbn1"])
    wg2, bg2 = _fold(p["g2"], p["gbn2"])
    we1, be1 = _fold(p["e1"], p["ebn1"])
    we2, be2 = _fold(p["e2"], p["ebn2"])
    we3, be3 = _fold(p["e3"], p["ebn3"])
    wn = p["arc_W"] / jnp.clip(
        jnp.linalg.norm(p["arc_W"], axis=1, keepdims=True), 1e-12, None)
    ws = [wg1.T, bg1[None, :], wg2.T, bg2[None, :],
          we1.T[:256], we1.T[256:], be1[None, :],
          we2.T, be2[None, :], we3.T, be3[None, :], wn.T]
    return _head(x1, x2, x3, ws)


# speedup vs baseline: 1.2814x; 1.2814x over previous
import functools

import jax
import jax.numpy as jnp
from jax import lax
from jax.experimental import pallas as pl
from jax.experimental.pallas import tpu as pltpu
from jax.experimental.pallas import tpu_sc as plsc

N = 8192
K = 20
R = 256
C = 1024
NCT = N // C
PB_E = 256
PB_H = 512
INT_MAX = 2**31 - 1


def _knn_body(xr_ref, xct_ref, br_ref, bc_ref, wdt_ref, wvt_ref, bdv_ref,
              idx_ref, u_ref, v_ref, bd_scr, bi_scr):
    j = pl.program_id(1)
    xr = xr_ref[...]

    @pl.when(j == 0)
    def _init():
        bd_scr[...] = jnp.full((R, 32), jnp.inf, jnp.float32)
        bi_scr[...] = jnp.zeros((R, 32), jnp.int32)
        u_ref[...] = (jnp.dot(xr, wdt_ref[...],
                              preferred_element_type=jnp.float32)
                      + bdv_ref[...])
        v_ref[...] = jnp.dot(xr, wvt_ref[...],
                             preferred_element_type=jnp.float32)

    br = br_ref[...]
    bc = bc_ref[...]
    valid = (jnp.min(bc) <= jnp.max(br)) & (jnp.max(bc) >= jnp.min(br))

    @pl.when(valid)
    def _tile():
        xct = xct_ref[...]
        sqc = jnp.sum(xct * xct, axis=0, keepdims=True)
        sqr = jnp.sum(xr * xr, axis=1, keepdims=True)
        g = lax.dot_general(xr, xct, (((1,), (0,)), ((), ())),
                            preferred_element_type=jnp.float32)
        rank = (sqr + sqc) - 2.0 * g
        mask = br != bc
        rank = jnp.where(mask, jnp.inf, rank)
        ids = lax.broadcasted_iota(jnp.int32, (R, C), 1) + j * C
        cur = jnp.concatenate([rank, bd_scr[...]], axis=1)
        idc = jnp.concatenate([ids, bi_scr[...]], axis=1)
        ms, mis = [], []
        for _ in range(K):
            m = jnp.min(cur, axis=1, keepdims=True)
            sel = jnp.where(cur == m, idc, INT_MAX)
            mi = jnp.min(sel, axis=1, keepdims=True)
            ms.append(m)
            mis.append(mi)
            cur = jnp.where(sel == mi, jnp.inf, cur)
        bd_scr[:, 0:K] = jnp.concatenate(ms, axis=1)
        bi_scr[:, 0:K] = jnp.concatenate(mis, axis=1)

    @pl.when(j == NCT - 1)
    def _fin():
        idx_ref[...] = bi_scr[:, 0:K]


def _knn(x, batch2d_r, batch2d_c, wdt, wvt, bdv):
    f = x.shape[1]
    fu = wdt.shape[1]
    return pl.pallas_call(
        _knn_body,
        grid=(N // R, NCT),
        in_specs=[
            pl.BlockSpec((R, f), lambda i, j: (i, 0)),
            pl.BlockSpec((f, C), lambda i, j: (0, j)),
            pl.BlockSpec((R, 1), lambda i, j: (i, 0)),
            pl.BlockSpec((1, C), lambda i, j: (0, j)),
            pl.BlockSpec((f, fu), lambda i, j: (0, 0)),
            pl.BlockSpec((f, fu), lambda i, j: (0, 0)),
            pl.BlockSpec((1, fu), lambda i, j: (0, 0)),
        ],
        out_specs=[
            pl.BlockSpec((R, K), lambda i, j: (i, 0)),
            pl.BlockSpec((R, fu), lambda i, j: (i, 0)),
            pl.BlockSpec((R, fu), lambda i, j: (i, 0)),
        ],
        out_shape=[
            jax.ShapeDtypeStruct((N, K), jnp.int32),
            jax.ShapeDtypeStruct((N, fu), jnp.float32),
            jax.ShapeDtypeStruct((N, fu), jnp.float32),
        ],
        scratch_shapes=[
            pltpu.VMEM((R, 32), jnp.float32),
            pltpu.VMEM((R, 32), jnp.int32),
        ],
    )(x, x.T, batch2d_r, batch2d_c, wdt, wvt, bdv)


def _sc_gather(table, idx_flat):
    f = table.shape[1]
    e = idx_flat.shape[0]
    nw = 32
    e_per_w = e // nw
    ch = 128
    n_ch = e_per_w // ch
    mesh = plsc.VectorSubcoreMesh(core_axis_name="c", subcore_axis_name="s")

    @functools.partial(
        pl.kernel, mesh=mesh,
        compiler_params=pltpu.CompilerParams(use_tc_tiling_on_sc=False),
        out_type=jax.ShapeDtypeStruct((e, f), jnp.float32),
        scratch_types=[
            pltpu.VMEM((e_per_w,), jnp.int32),
            pltpu.VMEM((ch, f), jnp.float32),
            pltpu.VMEM((ch, f), jnp.float32),
            pltpu.SemaphoreType.DMA,
            pltpu.SemaphoreType.DMA,
            pltpu.SemaphoreType.DMA,
            pltpu.SemaphoreType.DMA,
        ],
    )
    def k(table_hbm, idx_hbm, out_hbm, idx_v, rows0, rows1,
          gs0, gs1, ws0, ws1):
        wid = lax.axis_index("s") * 2 + lax.axis_index("c")
        base = wid * e_per_w
        pltpu.sync_copy(idx_hbm.at[pl.ds(base, e_per_w)], idx_v)
        rows = (rows0, rows1)
        gs = (gs0, gs1)
        ws = (ws0, ws1)

        def gcopy(ci, p):
            return pltpu.make_async_copy(
                table_hbm.at[idx_v.at[pl.ds(ci * ch, ch)]], rows[p], gs[p])

        def wcopy(ci, p):
            return pltpu.make_async_copy(
                rows[p], out_hbm.at[pl.ds(base + ci * ch, ch)], ws[p])

        def body(k2, carry):
            ci0 = k2 * 2
            ci1 = ci0 + 1

            @pl.when(ci0 >= 2)
            def _():
                wcopy(ci0 - 2, 0).wait()

            gcopy(ci0, 0).start()

            @pl.when(ci0 >= 1)
            def _():
                gcopy(ci0 - 1, 1).wait()
                wcopy(ci0 - 1, 1).start()

            @pl.when(ci1 >= 2)
            def _():
                wcopy(ci1 - 2, 1).wait()

            gcopy(ci1, 1).start()
            gcopy(ci0, 0).wait()
            wcopy(ci0, 0).start()
            return carry

        lax.fori_loop(0, n_ch // 2, body, 0)
        gcopy(n_ch - 1, 1).wait()
        wcopy(n_ch - 1, 1).start()
        wcopy(n_ch - 2, 0).wait()
        wcopy(n_ch - 1, 1).wait()

    return k(table, idx_flat)


def _edge_body(vg_ref, u_ref, w2_ref, b2_ref, out_ref):
    u = u_ref[...]
    w2 = w2_ref[...]
    b2 = b2_ref[...]
    f = u.shape[1]
    acc = jnp.full((PB_E, f), -jnp.inf, jnp.float32)
    for kk in range(K):
        vk = vg_ref[:, kk, :]
        h1 = jnp.maximum(u + vk, 0.0)
        h2 = jnp.dot(h1, w2, preferred_element_type=jnp.float32) + b2
        acc = jnp.maximum(acc, h2)
    out_ref[...] = jnp.maximum(acc, 0.0)


def _edge_mlp(vg3, u, w2t, b2v):
    f = u.shape[1]
    return pl.pallas_call(
        _edge_body,
        grid=(N // PB_E,),
        in_specs=[
            pl.BlockSpec((PB_E, K, f), lambda i: (i, 0, 0)),
            pl.BlockSpec((PB_E, f), lambda i: (i, 0)),
            pl.BlockSpec((f, f), lambda i: (0, 0)),
            pl.BlockSpec((1, f), lambda i: (0, 0)),
        ],
        out_specs=pl.BlockSpec((PB_E, f), lambda i: (i, 0)),
        out_shape=jax.ShapeDtypeStruct((N, f), jnp.float32),
    )(vg3, u, w2t, b2v)


def _head_body(x1_ref, x2_ref, x3_ref, wg1_ref, bg1_ref, wg2_ref, bg2_ref,
               we1a_ref, we1b_ref, be1_ref, we2_ref, be2_ref, we3_ref,
               be3_ref, wn_ref, out_ref):
    c = jnp.concatenate([x1_ref[...], x2_ref[...], x3_ref[...]], axis=1)
    g1 = jnp.maximum(jnp.dot(c, wg1_ref[...],
                             preferred_element_type=jnp.float32)
                     + bg1_ref[...], 0.0)
    g2 = jnp.maximum(jnp.dot(g1, wg2_ref[...],
                             preferred_element_type=jnp.float32)
                     + bg2_ref[...], 0.0)
    e1 = jnp.maximum(jnp.dot(c, we1a_ref[...],
                             preferred_element_type=jnp.float32)
                     + jnp.dot(g2, we1b_ref[...],
                               preferred_element_type=jnp.float32)
                     + be1_ref[...], 0.0)
    e2 = jnp.maximum(jnp.dot(e1, we2_ref[...],
                             preferred_element_type=jnp.float32)
                     + be2_ref[...], 0.0)
    h = jnp.dot(e2, we3_ref[...], preferred_element_type=jnp.float32) \
        + be3_ref[...]
    nrm = jnp.sqrt(jnp.sum(h * h, axis=1, keepdims=True))
    xn = h / jnp.maximum(nrm, 1e-12)
    out_ref[...] = jnp.dot(xn, wn_ref[...],
                           preferred_element_type=jnp.float32) * 30.0


def _head(x1, x2, x3, ws):
    full = lambda a: pl.BlockSpec(a.shape, lambda i: tuple(0 for _ in a.shape))
    return pl.pallas_call(
        _head_body,
        grid=(N // PB_H,),
        in_specs=[
            pl.BlockSpec((PB_H, 64), lambda i: (i, 0)),
            pl.BlockSpec((PB_H, 64), lambda i: (i, 0)),
            pl.BlockSpec((PB_H, 128), lambda i: (i, 0)),
        ] + [full(w) for w in ws],
        out_specs=pl.BlockSpec((PB_H, 3), lambda i: (i, 0)),
        out_shape=jax.ShapeDtypeStruct((N, 3), jnp.float32),
    )(x1, x2, x3, *ws)


_BN_C = 1.0 / float(1.0 + 1e-5) ** 0.5


def _fold(lin, bn):
    s = bn["g"] * _BN_C
    return lin["W"] * s[:, None], lin["b"] * s + bn["bt"]


def _edge_params(l1, bn1, l2, bn2, fin):
    w1, b1 = _fold(l1, bn1)
    wa, wb = w1[:, :fin], w1[:, fin:]
    wdt = (wa - wb).T
    wvt = wb.T
    w2, b2 = _fold(l2, bn2)
    return wdt, wvt, b1[None, :], w2.T, b2[None, :]


def kernel(pos, batch, label, params):
    p = params
    br = batch.reshape(N, 1)
    bc = batch.reshape(1, N)

    wdt1, wvt1, bd1, w2t1, b2v1 = _edge_params(
        p["c1l1"], p["c1bn1"], p["c1l2"], p["c1bn2"], 3)
    wdt2, wvt2, bd2, w2t2, b2v2 = _edge_params(
        p["c2l1"], p["c2bn1"], p["c2l2"], p["c2bn2"], 64)
    wdt3, wvt3, bd3, w2t3, b2v3 = _edge_params(
        p["c3l1"], p["c3bn1"], p["c3l2"], p["c3bn2"], 64)

    idx1, u1, v1 = _knn(pos, br, bc, wdt1, wvt1, bd1)
    vg1 = _sc_gather(v1, idx1.reshape(-1)).reshape(N, K, 64)
    x1 = _edge_mlp(vg1, u1, w2t1, b2v1)

    idx2, u2, v2 = _knn(x1, br, bc, wdt2, wvt2, bd2)
    vg2 = _sc_gather(v2, idx2.reshape(-1)).reshape(N, K, 64)
    x2 = _edge_mlp(vg2, u2, w2t2, b2v2)

    idx3, u3, v3 = _knn(x2, br, bc, wdt3, wvt3, bd3)
    vg3 = _sc_gather(v3, idx3.reshape(-1)).reshape(N, K, 128)
    x3 = _edge_mlp(vg3, u3, w2t3, b2v3)

    wg1, bg1 = _fold(p["g1"], p["gbn1"])
    wg2, bg2 = _fold(p["g2"], p["gbn2"])
    we1, be1 = _fold(p["e1"], p["ebn1"])
    we2, be2 = _fold(p["e2"], p["ebn2"])
    we3, be3 = _fold(p["e3"], p["ebn3"])
    wn = p["arc_W"] / jnp.clip(
        jnp.linalg.norm(p["arc_W"], axis=1, keepdims=True), 1e-12, None)
    ws = [wg1.T, bg1[None, :], wg2.T, bg2[None, :],
          we1.T[:256], we1.T[256:], be1[None, :],
          we2.T, be2[None, :], we3.T, be3[None, :], wn.T]
    return _head(x1, x2, x3, ws)


# speedup vs baseline: 1.5727x; 1.2273x over previous
import functools

import jax
import jax.numpy as jnp
from jax import lax
from jax.experimental import pallas as pl
from jax.experimental.pallas import tpu as pltpu
from jax.experimental.pallas import tpu_sc as plsc

N = 8192
K = 20
R = 256
C = 1024
NCT = N // C
PB_E = 256
PB_H = 512
BIG_ID = 3.0e38


def _knn_body(xr_ref, xct_ref, br_ref, bc_ref, wdt_ref, wvt_ref, bdv_ref,
              idx_ref, u_ref, v_ref, bd_scr, bi_scr):
    j = pl.program_id(1)
    xr = xr_ref[...]

    @pl.when(j == 0)
    def _init():
        bd_scr[...] = jnp.full((R, 32), jnp.inf, jnp.float32)
        bi_scr[...] = jnp.zeros((R, 32), jnp.float32)
        u_ref[...] = (jnp.dot(xr, wdt_ref[...],
                              preferred_element_type=jnp.float32)
                      + bdv_ref[...])
        v_ref[...] = jnp.dot(xr, wvt_ref[...],
                             preferred_element_type=jnp.float32)

    br = br_ref[...]
    bc = bc_ref[...]
    valid = (jnp.min(bc) <= jnp.max(br)) & (jnp.max(bc) >= jnp.min(br))

    @pl.when(valid)
    def _tile():
        xct = xct_ref[...]
        sqc = jnp.sum(xct * xct, axis=0, keepdims=True)
        sqr = jnp.sum(xr * xr, axis=1, keepdims=True)
        g = lax.dot_general(xr, xct, (((1,), (0,)), ((), ())),
                            preferred_element_type=jnp.float32)
        rank = (sqr + sqc) - 2.0 * g
        mask = br != bc
        rank = jnp.where(mask, jnp.inf, rank)
        ids = (lax.broadcasted_iota(jnp.int32, (R, C), 1).astype(jnp.float32)
               + jnp.float32(j * C))
        cur = jnp.concatenate([rank, bd_scr[...]], axis=1)
        idc = jnp.concatenate([ids, bi_scr[...]], axis=1)
        ms, mis = [], []
        for _ in range(K):
            m = jnp.min(cur, axis=1, keepdims=True)
            sel = jnp.where(cur == m, idc, jnp.float32(BIG_ID))
            mi = jnp.min(sel, axis=1, keepdims=True)
            ms.append(m)
            mis.append(mi)
            cur = jnp.where(sel == mi, jnp.inf, cur)
        bd_scr[:, 0:K] = jnp.concatenate(ms, axis=1)
        bi_scr[:, 0:K] = jnp.concatenate(mis, axis=1)

    @pl.when(j == NCT - 1)
    def _fin():
        idx_ref[...] = bi_scr[:, 0:K].astype(jnp.int32)


def _knn(x, batch2d_r, batch2d_c, wdt, wvt, bdv):
    f = x.shape[1]
    fu = wdt.shape[1]
    return pl.pallas_call(
        _knn_body,
        grid=(N // R, NCT),
        in_specs=[
            pl.BlockSpec((R, f), lambda i, j: (i, 0)),
            pl.BlockSpec((f, C), lambda i, j: (0, j)),
            pl.BlockSpec((R, 1), lambda i, j: (i, 0)),
            pl.BlockSpec((1, C), lambda i, j: (0, j)),
            pl.BlockSpec((f, fu), lambda i, j: (0, 0)),
            pl.BlockSpec((f, fu), lambda i, j: (0, 0)),
            pl.BlockSpec((1, fu), lambda i, j: (0, 0)),
        ],
        out_specs=[
            pl.BlockSpec((R, K), lambda i, j: (i, 0)),
            pl.BlockSpec((R, fu), lambda i, j: (i, 0)),
            pl.BlockSpec((R, fu), lambda i, j: (i, 0)),
        ],
        out_shape=[
            jax.ShapeDtypeStruct((N, K), jnp.int32),
            jax.ShapeDtypeStruct((N, fu), jnp.float32),
            jax.ShapeDtypeStruct((N, fu), jnp.float32),
        ],
        scratch_shapes=[
            pltpu.VMEM((R, 32), jnp.float32),
            pltpu.VMEM((R, 32), jnp.float32),
        ],
    )(x, x.T, batch2d_r, batch2d_c, wdt, wvt, bdv)


def _sc_gather(table, idx_flat):
    f = table.shape[1]
    e = idx_flat.shape[0]
    nw = 32
    e_per_w = e // nw
    ch = 128
    n_ch = e_per_w // ch
    mesh = plsc.VectorSubcoreMesh(core_axis_name="c", subcore_axis_name="s")

    @functools.partial(
        pl.kernel, mesh=mesh,
        compiler_params=pltpu.CompilerParams(use_tc_tiling_on_sc=False),
        out_type=jax.ShapeDtypeStruct((e, f), jnp.float32),
        scratch_types=[
            pltpu.VMEM((e_per_w,), jnp.int32),
            pltpu.VMEM((ch, f), jnp.float32),
            pltpu.VMEM((ch, f), jnp.float32),
            pltpu.SemaphoreType.DMA,
            pltpu.SemaphoreType.DMA,
            pltpu.SemaphoreType.DMA,
            pltpu.SemaphoreType.DMA,
        ],
    )
    def k(table_hbm, idx_hbm, out_hbm, idx_v, rows0, rows1,
          gs0, gs1, ws0, ws1):
        wid = lax.axis_index("s") * 2 + lax.axis_index("c")
        base = wid * e_per_w
        pltpu.sync_copy(idx_hbm.at[pl.ds(base, e_per_w)], idx_v)
        rows = (rows0, rows1)
        gs = (gs0, gs1)
        ws = (ws0, ws1)

        def gcopy(ci, p):
            return pltpu.make_async_copy(
                table_hbm.at[idx_v.at[pl.ds(ci * ch, ch)]], rows[p], gs[p])

        def wcopy(ci, p):
            return pltpu.make_async_copy(
                rows[p], out_hbm.at[pl.ds(base + ci * ch, ch)], ws[p])

        def body(k2, carry):
            ci0 = k2 * 2
            ci1 = ci0 + 1

            @pl.when(ci0 >= 2)
            def _():
                wcopy(ci0 - 2, 0).wait()

            gcopy(ci0, 0).start()

            @pl.when(ci0 >= 1)
            def _():
                gcopy(ci0 - 1, 1).wait()
                wcopy(ci0 - 1, 1).start()

            @pl.when(ci1 >= 2)
            def _():
                wcopy(ci1 - 2, 1).wait()

            gcopy(ci1, 1).start()
            gcopy(ci0, 0).wait()
            wcopy(ci0, 0).start()
            return carry

        lax.fori_loop(0, n_ch // 2, body, 0)
        gcopy(n_ch - 1, 1).wait()
        wcopy(n_ch - 1, 1).start()
        wcopy(n_ch - 2, 0).wait()
        wcopy(n_ch - 1, 1).wait()

    return k(table, idx_flat)


def _edge_body(vg_ref, u_ref, w2_ref, b2_ref, out_ref):
    u = u_ref[...]
    w2 = w2_ref[...]
    b2 = b2_ref[...]
    hs = [jnp.maximum(u + vg_ref[:, kk, :], 0.0) for kk in range(K)]
    hcat = jnp.concatenate(hs, axis=0)
    h2 = jnp.dot(hcat, w2, preferred_element_type=jnp.float32)
    acc = h2[0:PB_E]
    for kk in range(1, K):
        acc = jnp.maximum(acc, h2[kk * PB_E:(kk + 1) * PB_E])
    out_ref[...] = jnp.maximum(acc + b2, 0.0)


def _edge_mlp(vg3, u, w2t, b2v):
    f = u.shape[1]
    return pl.pallas_call(
        _edge_body,
        grid=(N // PB_E,),
        in_specs=[
            pl.BlockSpec((PB_E, K, f), lambda i: (i, 0, 0)),
            pl.BlockSpec((PB_E, f), lambda i: (i, 0)),
            pl.BlockSpec((f, f), lambda i: (0, 0)),
            pl.BlockSpec((1, f), lambda i: (0, 0)),
        ],
        out_specs=pl.BlockSpec((PB_E, f), lambda i: (i, 0)),
        out_shape=jax.ShapeDtypeStruct((N, f), jnp.float32),
    )(vg3, u, w2t, b2v)


def _head_body(x1_ref, x2_ref, x3_ref, wg1_ref, bg1_ref, wg2_ref, bg2_ref,
               we1a_ref, we1b_ref, be1_ref, we2_ref, be2_ref, we3_ref,
               be3_ref, wn_ref, out_ref):
    c = jnp.concatenate([x1_ref[...], x2_ref[...], x3_ref[...]], axis=1)
    g1 = jnp.maximum(jnp.dot(c, wg1_ref[...],
                             preferred_element_type=jnp.float32)
                     + bg1_ref[...], 0.0)
    g2 = jnp.maximum(jnp.dot(g1, wg2_ref[...],
                             preferred_element_type=jnp.float32)
                     + bg2_ref[...], 0.0)
    e1 = jnp.maximum(jnp.dot(c, we1a_ref[...],
                             preferred_element_type=jnp.float32)
                     + jnp.dot(g2, we1b_ref[...],
                               preferred_element_type=jnp.float32)
                     + be1_ref[...], 0.0)
    e2 = jnp.maximum(jnp.dot(e1, we2_ref[...],
                             preferred_element_type=jnp.float32)
                     + be2_ref[...], 0.0)
    h = jnp.dot(e2, we3_ref[...], preferred_element_type=jnp.float32) \
        + be3_ref[...]
    nrm = jnp.sqrt(jnp.sum(h * h, axis=1, keepdims=True))
    xn = h / jnp.maximum(nrm, 1e-12)
    out_ref[...] = jnp.dot(xn, wn_ref[...],
                           preferred_element_type=jnp.float32) * 30.0


def _head(x1, x2, x3, ws):
    full = lambda a: pl.BlockSpec(a.shape, lambda i: tuple(0 for _ in a.shape))
    return pl.pallas_call(
        _head_body,
        grid=(N // PB_H,),
        in_specs=[
            pl.BlockSpec((PB_H, 64), lambda i: (i, 0)),
            pl.BlockSpec((PB_H, 64), lambda i: (i, 0)),
            pl.BlockSpec((PB_H, 128), lambda i: (i, 0)),
        ] + [full(w) for w in ws],
        out_specs=pl.BlockSpec((PB_H, 3), lambda i: (i, 0)),
        out_shape=jax.ShapeDtypeStruct((N, 3), jnp.float32),
    )(x1, x2, x3, *ws)


_BN_C = 1.0 / float(1.0 + 1e-5) ** 0.5


def _fold(lin, bn):
    s = bn["g"] * _BN_C
    return lin["W"] * s[:, None], lin["b"] * s + bn["bt"]


def _edge_params(l1, bn1, l2, bn2, fin):
    w1, b1 = _fold(l1, bn1)
    wa, wb = w1[:, :fin], w1[:, fin:]
    wdt = (wa - wb).T
    wvt = wb.T
    w2, b2 = _fold(l2, bn2)
    return wdt, wvt, b1[None, :], w2.T, b2[None, :]


def kernel(pos, batch, label, params):
    p = params
    br = batch.reshape(N, 1)
    bc = batch.reshape(1, N)

    wdt1, wvt1, bd1, w2t1, b2v1 = _edge_params(
        p["c1l1"], p["c1bn1"], p["c1l2"], p["c1bn2"], 3)
    wdt2, wvt2, bd2, w2t2, b2v2 = _edge_params(
        p["c2l1"], p["c2bn1"], p["c2l2"], p["c2bn2"], 64)
    wdt3, wvt3, bd3, w2t3, b2v3 = _edge_params(
        p["c3l1"], p["c3bn1"], p["c3l2"], p["c3bn2"], 64)

    idx1, u1, v1 = _knn(pos, br, bc, wdt1, wvt1, bd1)
    vg1 = _sc_gather(v1, idx1.reshape(-1)).reshape(N, K, 64)
    x1 = _edge_mlp(vg1, u1, w2t1, b2v1)

    idx2, u2, v2 = _knn(x1, br, bc, wdt2, wvt2, bd2)
    vg2 = _sc_gather(v2, idx2.reshape(-1)).reshape(N, K, 64)
    x2 = _edge_mlp(vg2, u2, w2t2, b2v2)

    idx3, u3, v3 = _knn(x2, br, bc, wdt3, wvt3, bd3)
    vg3 = _sc_gather(v3, idx3.reshape(-1)).reshape(N, K, 128)
    x3 = _edge_mlp(vg3, u3, w2t3, b2v3)

    wg1, bg1 = _fold(p["g1"], p["gbn1"])
    wg2, bg2 = _fold(p["g2"], p["gbn2"])
    we1, be1 = _fold(p["e1"], p["ebn1"])
    we2, be2 = _fold(p["e2"], p["ebn2"])
    we3, be3 = _fold(p["e3"], p["ebn3"])
    wn = p["arc_W"] / jnp.clip(
        jnp.linalg.norm(p["arc_W"], axis=1, keepdims=True), 1e-12, None)
    ws = [wg1.T, bg1[None, :], wg2.T, bg2[None, :],
          we1.T[:256], we1.T[256:], be1[None, :],
          we2.T, be2[None, :], we3.T, be3[None, :], wn.T]
    return _head(x1, x2, x3, ws)


# speedup vs baseline: 2.0301x; 1.2909x over previous
import functools

import jax
import jax.numpy as jnp
from jax import lax
from jax.experimental import pallas as pl
from jax.experimental.pallas import tpu as pltpu
from jax.experimental.pallas import tpu_sc as plsc

N = 8192
K = 20
R = 256
C = 1024
NCT = N // C
PB_E = 256
PB_H = 512
BIG_ID = 3.0e38


def _knn_body(xr_ref, xct_ref, br_ref, bc_ref, wdt_ref, wvt_ref, bdv_ref,
              idx_ref, u_ref, v_ref, bd_scr, bi_scr):
    j = pl.program_id(1)
    xr = xr_ref[...]

    @pl.when(j == 0)
    def _init():
        bd_scr[...] = jnp.full((R, 32), jnp.inf, jnp.float32)
        bi_scr[...] = jnp.zeros((R, 32), jnp.float32)
        u_ref[...] = (jnp.dot(xr, wdt_ref[...],
                              preferred_element_type=jnp.float32)
                      + bdv_ref[...])
        v_ref[...] = jnp.dot(xr, wvt_ref[...],
                             preferred_element_type=jnp.float32)

    br = br_ref[...]
    bc = bc_ref[...]
    valid = (jnp.min(bc) <= jnp.max(br)) & (jnp.max(bc) >= jnp.min(br))

    @pl.when(valid)
    def _tile():
        xct = xct_ref[...]
        sqc = jnp.sum(xct * xct, axis=0, keepdims=True)
        sqr = jnp.sum(xr * xr, axis=1, keepdims=True)
        g = lax.dot_general(xr, xct, (((1,), (0,)), ((), ())),
                            preferred_element_type=jnp.float32)
        rank = (sqr + sqc) - 2.0 * g
        mask = br != bc
        rank = jnp.where(mask, jnp.inf, rank)
        ids = (lax.broadcasted_iota(jnp.int32, (R, C), 1).astype(jnp.float32)
               + jnp.float32(j * C))
        cur = jnp.concatenate([rank, bd_scr[...]], axis=1)
        idc = jnp.concatenate([ids, bi_scr[...]], axis=1)
        ms, mis = [], []
        for _ in range(K):
            m = jnp.min(cur, axis=1, keepdims=True)
            sel = jnp.where(cur == m, idc, jnp.float32(BIG_ID))
            mi = jnp.min(sel, axis=1, keepdims=True)
            ms.append(m)
            mis.append(mi)
            cur = jnp.where(sel == mi, jnp.inf, cur)
        bd_scr[:, 0:K] = jnp.concatenate(ms, axis=1)
        bi_scr[:, 0:K] = jnp.concatenate(mis, axis=1)

    @pl.when(j == NCT - 1)
    def _fin():
        idx_ref[...] = bi_scr[:, 0:K].astype(jnp.int32)


def _knn(x, batch2d_r, batch2d_c, wdt, wvt, bdv):
    f = x.shape[1]
    fu = wdt.shape[1]
    return pl.pallas_call(
        _knn_body,
        grid=(N // R, NCT),
        in_specs=[
            pl.BlockSpec((R, f), lambda i, j: (i, 0)),
            pl.BlockSpec((f, C), lambda i, j: (0, j)),
            pl.BlockSpec((R, 1), lambda i, j: (i, 0)),
            pl.BlockSpec((1, C), lambda i, j: (0, j)),
            pl.BlockSpec((f, fu), lambda i, j: (0, 0)),
            pl.BlockSpec((f, fu), lambda i, j: (0, 0)),
            pl.BlockSpec((1, fu), lambda i, j: (0, 0)),
        ],
        out_specs=[
            pl.BlockSpec((R, K), lambda i, j: (i, 0)),
            pl.BlockSpec((R, fu), lambda i, j: (i, 0)),
            pl.BlockSpec((R, fu), lambda i, j: (i, 0)),
        ],
        out_shape=[
            jax.ShapeDtypeStruct((N, K), jnp.int32),
            jax.ShapeDtypeStruct((N, fu), jnp.float32),
            jax.ShapeDtypeStruct((N, fu), jnp.float32),
        ],
        scratch_shapes=[
            pltpu.VMEM((R, 32), jnp.float32),
            pltpu.VMEM((R, 32), jnp.float32),
        ],
    )(x, x.T, batch2d_r, batch2d_c, wdt, wvt, bdv)


SLOT = 1280
NP = 8 * SLOT
RPS = SLOT // R


def _knn_packed_body(xr_ref, xct_ref, br_ref, bc_ref, wdt_ref, wvt_ref,
                     bdv_ref, idx_ref, u_ref, v_ref):
    xr = xr_ref[...]
    u_ref[...] = (jnp.dot(xr, wdt_ref[...],
                          preferred_element_type=jnp.float32)
                  + bdv_ref[...])
    v_ref[...] = jnp.dot(xr, wvt_ref[...],
                         preferred_element_type=jnp.float32)
    xct = xct_ref[...]
    sqc = jnp.sum(xct * xct, axis=0, keepdims=True)
    sqr = jnp.sum(xr * xr, axis=1, keepdims=True)
    g = lax.dot_general(xr, xct, (((1,), (0,)), ((), ())),
                        preferred_element_type=jnp.float32)
    rank = (sqr + sqc) - 2.0 * g
    cur = jnp.where(br_ref[...] != bc_ref[...], jnp.inf, rank)
    base = (pl.program_id(0) // RPS) * SLOT
    idc = (lax.broadcasted_iota(jnp.int32, (R, SLOT), 1).astype(jnp.float32)
           + base.astype(jnp.float32))
    mis = []
    for _ in range(K):
        m = jnp.min(cur, axis=1, keepdims=True)
        sel = jnp.where(cur == m, idc, jnp.float32(BIG_ID))
        mi = jnp.min(sel, axis=1, keepdims=True)
        mis.append(mi)
        cur = jnp.where(sel == mi, jnp.inf, cur)
    idx_ref[...] = jnp.concatenate(mis, axis=1).astype(jnp.int32)


def _knn_packed(x, batch2d_r, batch2d_c, wdt, wvt, bdv):
    f = x.shape[1]
    fu = wdt.shape[1]
    return pl.pallas_call(
        _knn_packed_body,
        grid=(NP // R,),
        in_specs=[
            pl.BlockSpec((R, f), lambda i: (i, 0)),
            pl.BlockSpec((f, SLOT), lambda i: (0, i // RPS)),
            pl.BlockSpec((R, 1), lambda i: (i, 0)),
            pl.BlockSpec((1, SLOT), lambda i: (0, i // RPS)),
            pl.BlockSpec((f, fu), lambda i: (0, 0)),
            pl.BlockSpec((f, fu), lambda i: (0, 0)),
            pl.BlockSpec((1, fu), lambda i: (0, 0)),
        ],
        out_specs=[
            pl.BlockSpec((R, K), lambda i: (i, 0)),
            pl.BlockSpec((R, fu), lambda i: (i, 0)),
            pl.BlockSpec((R, fu), lambda i: (i, 0)),
        ],
        out_shape=[
            jax.ShapeDtypeStruct((NP, K), jnp.int32),
            jax.ShapeDtypeStruct((NP, fu), jnp.float32),
            jax.ShapeDtypeStruct((NP, fu), jnp.float32),
        ],
    )(x, x.T, batch2d_r, batch2d_c, wdt, wvt, bdv)


def _sc_gather(table, idx_flat):
    f = table.shape[1]
    e = idx_flat.shape[0]
    nw = 32
    e_per_w = e // nw
    ch = 128
    n_ch = e_per_w // ch
    mesh = plsc.VectorSubcoreMesh(core_axis_name="c", subcore_axis_name="s")

    @functools.partial(
        pl.kernel, mesh=mesh,
        compiler_params=pltpu.CompilerParams(use_tc_tiling_on_sc=False),
        out_type=jax.ShapeDtypeStruct((e, f), jnp.float32),
        scratch_types=[
            pltpu.VMEM((e_per_w,), jnp.int32),
            pltpu.VMEM((ch, f), jnp.float32),
            pltpu.VMEM((ch, f), jnp.float32),
            pltpu.SemaphoreType.DMA,
            pltpu.SemaphoreType.DMA,
            pltpu.SemaphoreType.DMA,
            pltpu.SemaphoreType.DMA,
        ],
    )
    def k(table_hbm, idx_hbm, out_hbm, idx_v, rows0, rows1,
          gs0, gs1, ws0, ws1):
        wid = lax.axis_index("s") * 2 + lax.axis_index("c")
        base = wid * e_per_w
        pltpu.sync_copy(idx_hbm.at[pl.ds(base, e_per_w)], idx_v)
        rows = (rows0, rows1)
        gs = (gs0, gs1)
        ws = (ws0, ws1)

        def gcopy(ci, p):
            return pltpu.make_async_copy(
                table_hbm.at[idx_v.at[pl.ds(ci * ch, ch)]], rows[p], gs[p])

        def wcopy(ci, p):
            return pltpu.make_async_copy(
                rows[p], out_hbm.at[pl.ds(base + ci * ch, ch)], ws[p])

        def body(k2, carry):
            ci0 = k2 * 2
            ci1 = ci0 + 1

            @pl.when(ci0 >= 2)
            def _():
                wcopy(ci0 - 2, 0).wait()

            gcopy(ci0, 0).start()

            @pl.when(ci0 >= 1)
            def _():
                gcopy(ci0 - 1, 1).wait()
                wcopy(ci0 - 1, 1).start()

            @pl.when(ci1 >= 2)
            def _():
                wcopy(ci1 - 2, 1).wait()

            gcopy(ci1, 1).start()
            gcopy(ci0, 0).wait()
            wcopy(ci0, 0).start()
            return carry

        lax.fori_loop(0, n_ch // 2, body, 0)
        gcopy(n_ch - 1, 1).wait()
        wcopy(n_ch - 1, 1).start()
        wcopy(n_ch - 2, 0).wait()
        wcopy(n_ch - 1, 1).wait()

    return k(table, idx_flat)


def _edge_body(vg_ref, u_ref, w2_ref, b2_ref, out_ref):
    u = u_ref[...]
    w2 = w2_ref[...]
    b2 = b2_ref[...]
    hs = [jnp.maximum(u + vg_ref[:, kk, :], 0.0) for kk in range(K)]
    hcat = jnp.concatenate(hs, axis=0)
    h2 = jnp.dot(hcat, w2, preferred_element_type=jnp.float32)
    acc = h2[0:PB_E]
    for kk in range(1, K):
        acc = jnp.maximum(acc, h2[kk * PB_E:(kk + 1) * PB_E])
    out_ref[...] = jnp.maximum(acc + b2, 0.0)


def _edge_mlp(vg3, u, w2t, b2v):
    n, f = u.shape
    return pl.pallas_call(
        _edge_body,
        grid=(n // PB_E,),
        in_specs=[
            pl.BlockSpec((PB_E, K, f), lambda i: (i, 0, 0)),
            pl.BlockSpec((PB_E, f), lambda i: (i, 0)),
            pl.BlockSpec((f, f), lambda i: (0, 0)),
            pl.BlockSpec((1, f), lambda i: (0, 0)),
        ],
        out_specs=pl.BlockSpec((PB_E, f), lambda i: (i, 0)),
        out_shape=jax.ShapeDtypeStruct((n, f), jnp.float32),
    )(vg3, u, w2t, b2v)


def _head_body(x1_ref, x2_ref, x3_ref, wg1_ref, bg1_ref, wg2_ref, bg2_ref,
               we1a_ref, we1b_ref, be1_ref, we2_ref, be2_ref, we3_ref,
               be3_ref, wn_ref, out_ref):
    c = jnp.concatenate([x1_ref[...], x2_ref[...], x3_ref[...]], axis=1)
    g1 = jnp.maximum(jnp.dot(c, wg1_ref[...],
                             preferred_element_type=jnp.float32)
                     + bg1_ref[...], 0.0)
    g2 = jnp.maximum(jnp.dot(g1, wg2_ref[...],
                             preferred_element_type=jnp.float32)
                     + bg2_ref[...], 0.0)
    e1 = jnp.maximum(jnp.dot(c, we1a_ref[...],
                             preferred_element_type=jnp.float32)
                     + jnp.dot(g2, we1b_ref[...],
                               preferred_element_type=jnp.float32)
                     + be1_ref[...], 0.0)
    e2 = jnp.maximum(jnp.dot(e1, we2_ref[...],
                             preferred_element_type=jnp.float32)
                     + be2_ref[...], 0.0)
    h = jnp.dot(e2, we3_ref[...], preferred_element_type=jnp.float32) \
        + be3_ref[...]
    nrm = jnp.sqrt(jnp.sum(h * h, axis=1, keepdims=True))
    xn = h / jnp.maximum(nrm, 1e-12)
    out_ref[...] = jnp.dot(xn, wn_ref[...],
                           preferred_element_type=jnp.float32) * 30.0


def _head(x1, x2, x3, ws):
    n = x1.shape[0]
    full = lambda a: pl.BlockSpec(a.shape, lambda i: tuple(0 for _ in a.shape))
    return pl.pallas_call(
        _head_body,
        grid=(n // PB_H,),
        in_specs=[
            pl.BlockSpec((PB_H, 64), lambda i: (i, 0)),
            pl.BlockSpec((PB_H, 64), lambda i: (i, 0)),
            pl.BlockSpec((PB_H, 128), lambda i: (i, 0)),
        ] + [full(w) for w in ws],
        out_specs=pl.BlockSpec((PB_H, 3), lambda i: (i, 0)),
        out_shape=jax.ShapeDtypeStruct((n, 3), jnp.float32),
    )(x1, x2, x3, *ws)


_BN_C = 1.0 / float(1.0 + 1e-5) ** 0.5


def _fold(lin, bn):
    s = bn["g"] * _BN_C
    return lin["W"] * s[:, None], lin["b"] * s + bn["bt"]


def _edge_params(l1, bn1, l2, bn2, fin):
    w1, b1 = _fold(l1, bn1)
    wa, wb = w1[:, :fin], w1[:, fin:]
    wdt = (wa - wb).T
    wvt = wb.T
    w2, b2 = _fold(l2, bn2)
    return wdt, wvt, b1[None, :], w2.T, b2[None, :]


def kernel(pos, batch, label, params):
    p = params

    wdt1, wvt1, bd1, w2t1, b2v1 = _edge_params(
        p["c1l1"], p["c1bn1"], p["c1l2"], p["c1bn2"], 3)
    wdt2, wvt2, bd2, w2t2, b2v2 = _edge_params(
        p["c2l1"], p["c2bn1"], p["c2l2"], p["c2bn2"], 64)
    wdt3, wvt3, bd3, w2t3, b2v3 = _edge_params(
        p["c3l1"], p["c3bn1"], p["c3l2"], p["c3bn2"], 64)
    ep = ((wdt1, wvt1, bd1, w2t1, b2v1),
          (wdt2, wvt2, bd2, w2t2, b2v2),
          (wdt3, wvt3, bd3, w2t3, b2v3))

    def convs_general(_):
        br = batch.reshape(N, 1)
        bc = batch.reshape(1, N)
        x = pos
        xs = []
        for li in range(3):
            wdt, wvt, bdv, w2t, b2v = ep[li]
            idx, u, v = _knn(x, br, bc, wdt, wvt, bdv)
            f = w2t.shape[0]
            vg = _sc_gather(v, idx.reshape(-1)).reshape(N, K, f)
            x = _edge_mlp(vg, u, w2t, b2v)
            xs.append(x)
        return xs

    def convs_packed(_):
        starts = jnp.searchsorted(batch, jnp.arange(8, dtype=jnp.int32))
        perm = (batch * SLOT + jnp.arange(N, dtype=jnp.int32)
                - starts[batch]).astype(jnp.int32)
        posp = jnp.zeros((NP, 3), jnp.float32).at[perm].set(pos)
        bp = jnp.full((NP,), 8, jnp.int32).at[perm].set(batch)
        br = bp.reshape(NP, 1)
        bc = bp.reshape(1, NP)
        x = posp
        xs = []
        for li in range(3):
            wdt, wvt, bdv, w2t, b2v = ep[li]
            idx, u, v = _knn_packed(x, br, bc, wdt, wvt, bdv)
            f = w2t.shape[0]
            vg = _sc_gather(v, idx.reshape(-1)).reshape(NP, K, f)
            x = _edge_mlp(vg, u, w2t, b2v)
            xs.append(x)
        return [xi[perm] for xi in xs]

    sizes = jnp.diff(jnp.searchsorted(batch, jnp.arange(9, dtype=jnp.int32)))
    fits = jnp.max(sizes) <= SLOT
    x1, x2, x3 = lax.cond(fits, convs_packed, convs_general, 0)

    wg1, bg1 = _fold(p["g1"], p["gbn1"])
    wg2, bg2 = _fold(p["g2"], p["gbn2"])
    we1, be1 = _fold(p["e1"], p["ebn1"])
    we2, be2 = _fold(p["e2"], p["ebn2"])
    we3, be3 = _fold(p["e3"], p["ebn3"])
    wn = p["arc_W"] / jnp.clip(
        jnp.linalg.norm(p["arc_W"], axis=1, keepdims=True), 1e-12, None)
    ws = [wg1.T, bg1[None, :], wg2.T, bg2[None, :],
          we1.T[:256], we1.T[256:], be1[None, :],
          we2.T, be2[None, :], we3.T, be3[None, :], wn.T]
    return _head(x1, x2, x3, ws)
